# baseline (speedup 1.0000x reference)
import functools

import jax
import jax.numpy as jnp
from jax import lax
from jax.experimental import pallas as pl
from jax.experimental.pallas import tpu as pltpu
from jax.experimental.pallas import tpu_sc as plsc

_K = 16
_L = 16
_CS = 2048
_C = 1024
_U = 16
_BOOT = 32
_BIG = 2**30


def _sc_body(x_hbm, o_hbm, buf0, buf1, cand_i, heap_v, heap_i, tvec,
             obuf, sem0, sem1, *, nb, nd, units_per):
    li = lax.iota(jnp.int32, _L)
    wid = lax.axis_index("c") * 16 + lax.axis_index("s")

    def scan(buf, lo, hi, s0, cnt0):
        tv = tvec[...]
        zeros = jnp.zeros((_L,), jnp.int32)

        def _sklansky(vals):
            n = len(vals)
            if n == 1:
                return vals
            a = _sklansky(vals[: n // 2])
            bb = _sklansky(vals[n // 2:])
            return a + [a[-1] + x for x in bb]

        def step(g, cntx):
            base = g * _U
            sb = s0 + base
            sbx = sb * 65537
            ms = []
            incs = []
            for p in range(_U // 2):
                m = (
                    jnp.maximum(buf[base + 2 * p], buf[base + 2 * p + 1])
                    > tv
                )
                ms.append(m)
                inc = jnp.where(m, _L, 0)
                incs.append(cntx + inc if p == 0 else inc)
            pre = _sklansky(incs)
            for p in range(_U // 2):
                addr = cntx if p == 0 else pre[p - 1]
                packed = sbx + (2 * p + ((2 * p + 1) << 16))
                plsc.store_scatter(
                    cand_i,
                    [addr],
                    jnp.full((_L,), packed, jnp.int32),
                    mask=ms[p],
                )
            return pre[_U // 2 - 1]

        cntx = lax.fori_loop(
            lo // _U, hi // _U, step, cnt0 * _L + li
        )
        return (cntx - li) // _L

    def merges(cnt, buf, s0):
        mc = jnp.max(cnt)

        @pl.when(mc > 0)
        def _do():
            def insert(hv, hi, cv, ci):
                out_v, out_i = [], []
                for q in range(_K):
                    up = cv > hv[q]
                    t_v = jnp.where(up, cv, hv[q])
                    t_i = jnp.where(up, ci, hi[q])
                    if q == _K - 1:
                        out_v.append(t_v)
                        out_i.append(t_i)
                    else:
                        keep = (t_v < hv[q + 1]) | (
                            (t_v == hv[q + 1]) & (t_i > hi[q + 1])
                        )
                        out_v.append(jnp.where(keep, t_v, hv[q + 1]))
                        out_i.append(jnp.where(keep, t_i, hi[q + 1]))
                return out_v, out_i

            def round_r(r, carry):
                hv = list(carry[:_K])
                hi = list(carry[_K:])
                act = r < cnt
                packed = cand_i[pl.ds(r * _L, _L)]
                ra = jnp.bitwise_and(packed, 0xFFFF)
                rb = lax.shift_right_logical(packed, 16)
                va = plsc.load_gather(
                    buf, [jnp.where(act, ra - s0, 0), li], mask=act
                )
                vb = plsc.load_gather(
                    buf, [jnp.where(act, rb - s0, 0), li], mask=act
                )
                va = jnp.where(act, va, -jnp.inf)
                vb = jnp.where(act, vb, -jnp.inf)
                ia = jnp.where(act, ra, _BIG)
                ib = jnp.where(act, rb, _BIG)
                bgt = vb > va
                cv1 = jnp.where(bgt, vb, va)
                ci1 = jnp.where(bgt, ib, ia)
                cv2 = jnp.where(bgt, va, vb)
                ci2 = jnp.where(bgt, ia, ib)

                def work(c):
                    wv, wi = insert(list(c[:_K]), list(c[_K:]), cv1, ci1)

                    def second(c2):
                        sv, si = insert(
                            list(c2[:_K]), list(c2[_K:]), cv2, ci2
                        )
                        return tuple(sv) + tuple(si)

                    c2 = tuple(wv) + tuple(wi)
                    return lax.cond(
                        jnp.any(cv2 > wv[0]), second, lambda z: z, c2
                    )

                return lax.cond(
                    jnp.any(cv1 > hv[0]), work, lambda z: z, carry
                )

            init = tuple(
                heap_v[pl.ds(q * _L, _L)] for q in range(_K)
            ) + tuple(heap_i[pl.ds(q * _L, _L)] for q in range(_K))
            fin = lax.fori_loop(0, mc, round_r, init)
            for q in range(_K):
                heap_v[pl.ds(q * _L, _L)] = fin[q]
                heap_i[pl.ds(q * _L, _L)] = fin[_K + q]
            tvec[...] = fin[0]

    def unit_body(u, _):
        unit = wid * units_per + u
        b = unit // nd
        d0 = (unit % nd) * _L

        for q in range(_K):
            heap_v[pl.ds(q * _L, _L)] = jnp.full((_L,), -jnp.inf, jnp.float32)
            heap_i[pl.ds(q * _L, _L)] = jnp.full((_L,), _BIG, jnp.int32)
        tvec[...] = jnp.full((_L,), -jnp.inf, jnp.float32)

        def dma(g, buf, sem):
            return pltpu.make_async_copy(
                x_hbm.at[b, pl.ds(g * _CS, _CS), pl.ds(d0, _L)], buf, sem
            )

        def process(buf, g):
            s0 = g * _CS

            def boot(_):
                z = jnp.zeros((_L,), jnp.int32)
                merges(scan(buf, 0, _BOOT, s0, z), buf, s0)
                merges(scan(buf, _BOOT, 8 * _BOOT, s0, z), buf, s0)
                return scan(buf, 8 * _BOOT, _CS, s0, z)

            def plain(_):
                return scan(buf, 0, _CS, s0, jnp.zeros((_L,), jnp.int32))

            merges(lax.cond(g == 0, boot, plain, 0), buf, s0)

        dma(0, buf0, sem0).start()

        def pair_body(p, _):
            g0 = 2 * p
            dma(g0, buf0, sem0).wait()
            dma(g0 + 1, buf1, sem1).start()
            process(buf0, g0)
            dma(g0 + 1, buf1, sem1).wait()

            @pl.when(g0 + 2 < nb)
            def _next():
                dma(g0 + 2, buf0, sem0).start()

            process(buf1, g0 + 1)
            return 0

        lax.fori_loop(0, nb // 2, pair_body, 0)

        hv = [heap_v[pl.ds(q * _L, _L)] for q in range(_K)]
        hi = [heap_i[pl.ds(q * _L, _L)] for q in range(_K)]
        for row in range(_K):
            wv, wi = hv[0], hi[0]
            for q in range(1, _K):
                take = hi[q] < wi
                wv = jnp.where(take, hv[q], wv)
                wi = jnp.where(take, hi[q], wi)
            obuf[row] = wv
            if row < _K - 1:
                for q in range(_K):
                    used = hi[q] == wi
                    hi[q] = jnp.where(used, _BIG, hi[q])

        pltpu.sync_copy(obuf, o_hbm.at[b, :, pl.ds(d0, _L)])
        return 0

    lax.fori_loop(0, units_per, unit_body, 0)


def kernel(x):
    b, s, d = x.shape
    assert d % _L == 0 and s % _CS == 0 and (s // _CS) % 2 == 0
    nd = d // _L
    nb = s // _CS
    n_workers = 32
    units = b * nd
    assert units % n_workers == 0
    mesh = plsc.VectorSubcoreMesh(core_axis_name="c", subcore_axis_name="s")
    f = pl.kernel(
        functools.partial(
            _sc_body, nb=nb, nd=nd, units_per=units // n_workers
        ),
        out_type=jax.ShapeDtypeStruct((b, _K, d), jnp.float32),
        mesh=mesh,
        compiler_params=pltpu.CompilerParams(
            use_tc_tiling_on_sc=False, needs_layout_passes=False
        ),
        scratch_types=[
            pltpu.VMEM((_CS, _L), jnp.float32),
            pltpu.VMEM((_CS, _L), jnp.float32),
            pltpu.VMEM((_C * _L,), jnp.int32),
            pltpu.VMEM((_K * _L,), jnp.float32),
            pltpu.VMEM((_K * _L,), jnp.int32),
            pltpu.VMEM((_L,), jnp.float32),
            pltpu.VMEM((_K, _L), jnp.float32),
            pltpu.SemaphoreType.DMA,
            pltpu.SemaphoreType.DMA,
        ],
    )
    return f(x)


# speedup vs baseline: 1.1150x; 1.1150x over previous
import functools

import jax
import jax.numpy as jnp
from jax import lax
from jax.experimental import pallas as pl
from jax.experimental.pallas import tpu as pltpu
from jax.experimental.pallas import tpu_sc as plsc

_K = 16
_L = 16
_CS = 2048
_C = 1024
_U = 16
_BOOT = 32
_BIG = 2**30


def _sc_body(x_hbm, o_hbm, buf0, buf1, cand_i, heap_v, heap_i, tvec,
             obuf, sem0, sem1, *, nb, nd, units_per):
    li = lax.iota(jnp.int32, _L)
    wid = lax.axis_index("c") * 16 + lax.axis_index("s")

    def scan(buf, lo, hi, s0, cnt0):
        tv = tvec[...]
        zeros = jnp.zeros((_L,), jnp.int32)

        def _sklansky(vals):
            n = len(vals)
            if n == 1:
                return vals
            a = _sklansky(vals[: n // 2])
            bb = _sklansky(vals[n // 2:])
            return a + [a[-1] + x for x in bb]

        def step(g, cntx):
            base = g * _U
            sb = s0 + base
            sbx = sb * 65537
            ms = []
            incs = []
            for p in range(_U // 2):
                m = (
                    jnp.maximum(buf[base + 2 * p], buf[base + 2 * p + 1])
                    > tv
                )
                ms.append(m)
                inc = jnp.where(m, _L, 0)
                incs.append(cntx + inc if p == 0 else inc)
            pre = _sklansky(incs)
            for p in range(_U // 2):
                addr = cntx if p == 0 else pre[p - 1]
                packed = sbx + (2 * p + ((2 * p + 1) << 16))
                plsc.store_scatter(
                    cand_i,
                    [addr],
                    jnp.full((_L,), packed, jnp.int32),
                    mask=ms[p],
                )
            return pre[_U // 2 - 1]

        cntx = lax.fori_loop(
            lo // _U, hi // _U, step, cnt0 * _L + li
        )
        return (cntx - li) // _L

    def merges(cnt, buf, s0):
        mc = jnp.max(cnt)

        @pl.when(mc > 0)
        def _do():
            def insert(hv, hi, cv, ci):
                out_v, out_i = [], []
                for q in range(_K):
                    up = cv > hv[q]
                    t_v = jnp.where(up, cv, hv[q])
                    t_i = jnp.where(up, ci, hi[q])
                    if q == _K - 1:
                        out_v.append(t_v)
                        out_i.append(t_i)
                    else:
                        keep = (t_v < hv[q + 1]) | (
                            (t_v == hv[q + 1]) & (t_i > hi[q + 1])
                        )
                        out_v.append(jnp.where(keep, t_v, hv[q + 1]))
                        out_i.append(jnp.where(keep, t_i, hi[q + 1]))
                return out_v, out_i

            def round_r(r, carry):
                hv = list(carry[:_K])
                hi = list(carry[_K:])
                act = r < cnt
                packed = cand_i[pl.ds(r * _L, _L)]
                ra = jnp.bitwise_and(packed, 0xFFFF)
                rb = lax.shift_right_logical(packed, 16)
                va = plsc.load_gather(
                    buf, [jnp.where(act, ra - s0, 0), li], mask=act
                )
                vb = plsc.load_gather(
                    buf, [jnp.where(act, rb - s0, 0), li], mask=act
                )
                va = jnp.where(act, va, -jnp.inf)
                vb = jnp.where(act, vb, -jnp.inf)
                ia = jnp.where(act, ra, _BIG)
                ib = jnp.where(act, rb, _BIG)
                bgt = vb > va
                cv1 = jnp.where(bgt, vb, va)
                ci1 = jnp.where(bgt, ib, ia)
                cv2 = jnp.where(bgt, va, vb)
                ci2 = jnp.where(bgt, ia, ib)
                hv, hi = insert(hv, hi, cv1, ci1)

                def second(c):
                    sv, si = insert(list(c[:_K]), list(c[_K:]), cv2, ci2)
                    return tuple(sv) + tuple(si)

                carry2 = tuple(hv) + tuple(hi)
                return lax.cond(
                    jnp.any(cv2 > hv[0]), second, lambda c: c, carry2
                )

            init = tuple(
                heap_v[pl.ds(q * _L, _L)] for q in range(_K)
            ) + tuple(heap_i[pl.ds(q * _L, _L)] for q in range(_K))
            fin = lax.fori_loop(0, mc, round_r, init)
            for q in range(_K):
                heap_v[pl.ds(q * _L, _L)] = fin[q]
                heap_i[pl.ds(q * _L, _L)] = fin[_K + q]
            tvec[...] = fin[0]

    def unit_body(u, _):
        unit = wid * units_per + u
        b = unit // nd
        d0 = (unit % nd) * _L

        for q in range(_K):
            heap_v[pl.ds(q * _L, _L)] = jnp.full((_L,), -jnp.inf, jnp.float32)
            heap_i[pl.ds(q * _L, _L)] = jnp.full((_L,), _BIG, jnp.int32)
        tvec[...] = jnp.full((_L,), -jnp.inf, jnp.float32)

        def dma(g, buf, sem):
            return pltpu.make_async_copy(
                x_hbm.at[b, pl.ds(g * _CS, _CS), pl.ds(d0, _L)], buf, sem
            )

        def process(buf, g):
            s0 = g * _CS

            def boot(_):
                z = jnp.zeros((_L,), jnp.int32)
                merges(scan(buf, 0, _BOOT, s0, z), buf, s0)
                merges(scan(buf, _BOOT, 8 * _BOOT, s0, z), buf, s0)
                return scan(buf, 8 * _BOOT, _CS, s0, z)

            def plain(_):
                return scan(buf, 0, _CS, s0, jnp.zeros((_L,), jnp.int32))

            merges(lax.cond(g == 0, boot, plain, 0), buf, s0)

        dma(0, buf0, sem0).start()

        def pair_body(p, _):
            g0 = 2 * p
            dma(g0, buf0, sem0).wait()
            dma(g0 + 1, buf1, sem1).start()
            process(buf0, g0)
            dma(g0 + 1, buf1, sem1).wait()

            @pl.when(g0 + 2 < nb)
            def _next():
                dma(g0 + 2, buf0, sem0).start()

            process(buf1, g0 + 1)
            return 0

        lax.fori_loop(0, nb // 2, pair_body, 0)

        hv = [heap_v[pl.ds(q * _L, _L)] for q in range(_K)]
        hi = [heap_i[pl.ds(q * _L, _L)] for q in range(_K)]
        for row in range(_K):
            wv, wi = hv[0], hi[0]
            for q in range(1, _K):
                take = hi[q] < wi
                wv = jnp.where(take, hv[q], wv)
                wi = jnp.where(take, hi[q], wi)
            obuf[row] = wv
            if row < _K - 1:
                for q in range(_K):
                    used = hi[q] == wi
                    hi[q] = jnp.where(used, _BIG, hi[q])

        pltpu.sync_copy(obuf, o_hbm.at[b, :, pl.ds(d0, _L)])
        return 0

    lax.fori_loop(0, units_per, unit_body, 0)


def kernel(x):
    b, s, d = x.shape
    assert d % _L == 0 and s % _CS == 0 and (s // _CS) % 2 == 0
    nd = d // _L
    nb = s // _CS
    n_workers = 32
    units = b * nd
    assert units % n_workers == 0
    mesh = plsc.VectorSubcoreMesh(core_axis_name="c", subcore_axis_name="s")
    f = pl.kernel(
        functools.partial(
            _sc_body, nb=nb, nd=nd, units_per=units // n_workers
        ),
        out_type=jax.ShapeDtypeStruct((b, _K, d), jnp.float32),
        mesh=mesh,
        compiler_params=pltpu.CompilerParams(
            use_tc_tiling_on_sc=False, needs_layout_passes=False
        ),
        scratch_types=[
            pltpu.VMEM((_CS, _L), jnp.float32),
            pltpu.VMEM((_CS, _L), jnp.float32),
            pltpu.VMEM((_C * _L,), jnp.int32),
            pltpu.VMEM((_K * _L,), jnp.float32),
            pltpu.VMEM((_K * _L,), jnp.int32),
            pltpu.VMEM((_L,), jnp.float32),
            pltpu.VMEM((_K, _L), jnp.float32),
            pltpu.SemaphoreType.DMA,
            pltpu.SemaphoreType.DMA,
        ],
    )
    return f(x)
